# baseline (device time: 82804 ns/iter reference)
import jax
import jax.numpy as jnp
from jax import lax
from jax.experimental import pallas as pl
from jax.experimental.pallas import tpu as pltpu

M, N = 2048, 1024
H = 512
Q = 256


def kernel(x):
    def body(x4_ref, out_ref, xAv, xBv, accA, accB, rA0, rA1, rB0, rB1,
             send_sems, recv_sems, local_sems):
        x_ref = x4_ref.at[0, 0]
        my_x = lax.axis_index("x")
        my_y = lax.axis_index("y")
        y_nbr = (my_x, 1 - my_y)
        x_nbr = (1 - my_x, my_y)

        barrier_sem = pltpu.get_barrier_semaphore()
        for nbr in (y_nbr, x_nbr):
            pl.semaphore_signal(
                barrier_sem, inc=1,
                device_id=nbr, device_id_type=pl.DeviceIdType.MESH,
            )
        pl.semaphore_wait(barrier_sem, 2)

        def xfer(i, src, dst, nbr):
            return pltpu.make_async_remote_copy(
                src_ref=src, dst_ref=dst,
                send_sem=send_sems.at[i], recv_sem=recv_sems.at[i],
                device_id=nbr, device_id_type=pl.DeviceIdType.MESH,
            )

        aH_mine = my_y * H
        aH_peer = (1 - my_y) * H
        bH_mine = 1024 + my_x * H
        bH_peer = 1024 + (1 - my_x) * H
        fA = (1 - my_x) * Q
        sA = my_x * Q
        fB = (1 - my_y) * Q
        sB = my_y * Q

        a0f = xfer(0, x_ref.at[pl.ds(aH_peer + fA, Q), :],
                   rA0.at[pl.ds(fA, Q), :], y_nbr)
        a0s = xfer(1, x_ref.at[pl.ds(aH_peer + sA, Q), :],
                   rA0.at[pl.ds(sA, Q), :], y_nbr)
        b0f = xfer(2, x_ref.at[pl.ds(bH_peer + fB, Q), :],
                   rB0.at[pl.ds(fB, Q), :], x_nbr)
        b0s = xfer(3, x_ref.at[pl.ds(bH_peer + sB, Q), :],
                   rB0.at[pl.ds(sB, Q), :], x_nbr)
        a0f.start()
        a0s.start()
        b0f.start()
        b0s.start()
        inA = pltpu.make_async_copy(
            x_ref.at[pl.ds(aH_mine, H), :], xAv, local_sems.at[0])
        inB = pltpu.make_async_copy(
            x_ref.at[pl.ds(bH_mine, H), :], xBv, local_sems.at[1])
        inA.start()
        inB.start()

        a1 = xfer(4, accA.at[pl.ds(fA, Q), :], rA1, x_nbr)
        b1 = xfer(5, accB.at[pl.ds(fB, Q), :], rB1, y_nbr)
        inA.wait()
        a0f.wait_recv()
        accA[pl.ds(fA, Q), :] = xAv[pl.ds(fA, Q), :] + rA0[pl.ds(fA, Q), :]
        a1.start()
        inB.wait()
        b0f.wait_recv()
        accB[pl.ds(fB, Q), :] = xBv[pl.ds(fB, Q), :] + rB0[pl.ds(fB, Q), :]
        b1.start()

        a2 = xfer(6, accA.at[pl.ds(sA, Q), :],
                  accA.at[pl.ds(sA, Q), :], x_nbr)
        b2 = xfer(7, accB.at[pl.ds(sB, Q), :],
                  accB.at[pl.ds(sB, Q), :], y_nbr)
        a0s.wait_recv()
        a1.wait_recv()
        accA[pl.ds(sA, Q), :] = (
            xAv[pl.ds(sA, Q), :] + rA0[pl.ds(sA, Q), :] + rA1[...]
        )
        a2.start()
        b0s.wait_recv()
        b1.wait_recv()
        accB[pl.ds(sB, Q), :] = (
            xBv[pl.ds(sB, Q), :] + rB0[pl.ds(sB, Q), :] + rB1[...]
        )
        b2.start()

        a3 = xfer(8, accA, out_ref.at[pl.ds(aH_mine, H), :], y_nbr)
        b3 = xfer(9, accB, out_ref.at[pl.ds(bH_mine, H), :], x_nbr)
        outA = pltpu.make_async_copy(
            accA, out_ref.at[pl.ds(aH_mine, H), :], local_sems.at[2])
        outB = pltpu.make_async_copy(
            accB, out_ref.at[pl.ds(bH_mine, H), :], local_sems.at[3])
        a2.wait_recv()
        a3.start()
        outA.start()
        b2.wait_recv()
        b3.start()
        outB.start()

        a3.wait_recv()
        b3.wait_recv()
        outA.wait()
        outB.wait()
        for r in (a0f, a0s, b0f, b0s, a1, b1, a2, b2, a3, b3):
            r.wait_send()

    return pl.pallas_call(
        body,
        out_shape=jax.ShapeDtypeStruct((M, N), jnp.float32),
        in_specs=[pl.BlockSpec(memory_space=pl.ANY)],
        out_specs=pl.BlockSpec(memory_space=pl.ANY),
        scratch_shapes=[
            pltpu.VMEM((H, N), jnp.float32),
            pltpu.VMEM((H, N), jnp.float32),
            pltpu.VMEM((H, N), jnp.float32),
            pltpu.VMEM((H, N), jnp.float32),
            pltpu.VMEM((H, N), jnp.float32),
            pltpu.VMEM((Q, N), jnp.float32),
            pltpu.VMEM((H, N), jnp.float32),
            pltpu.VMEM((Q, N), jnp.float32),
            pltpu.SemaphoreType.DMA((10,)),
            pltpu.SemaphoreType.DMA((10,)),
            pltpu.SemaphoreType.DMA((4,)),
        ],
        compiler_params=pltpu.CompilerParams(collective_id=0),
    )(x)


# device time: 80506 ns/iter; 1.0285x vs baseline; 1.0285x over previous
import jax
import jax.numpy as jnp
from jax import lax
from jax.experimental import pallas as pl
from jax.experimental.pallas import tpu as pltpu

M, N = 2048, 1024
H = 512
Q = 256
E = 128


def kernel(x):
    def body(x4_ref, out_ref, xAv, xBv, accA, accB, rA0, rA1, rB0, rB1,
             send_sems, recv_sems, local_sems):
        x_ref = x4_ref.at[0, 0]
        my_x = lax.axis_index("x")
        my_y = lax.axis_index("y")
        y_nbr = (my_x, 1 - my_y)
        x_nbr = (1 - my_x, my_y)

        barrier_sem = pltpu.get_barrier_semaphore()
        for nbr in (y_nbr, x_nbr):
            pl.semaphore_signal(
                barrier_sem, inc=1,
                device_id=nbr, device_id_type=pl.DeviceIdType.MESH,
            )
        pl.semaphore_wait(barrier_sem, 2)

        def xfer(i, src, dst, nbr):
            return pltpu.make_async_remote_copy(
                src_ref=src, dst_ref=dst,
                send_sem=send_sems.at[i], recv_sem=recv_sems.at[i],
                device_id=nbr, device_id_type=pl.DeviceIdType.MESH,
            )

        aH_mine = my_y * H
        aH_peer = (1 - my_y) * H
        bH_mine = 1024 + my_x * H
        bH_peer = 1024 + (1 - my_x) * H
        fA = (1 - my_x) * Q
        sA = my_x * Q
        fB = (1 - my_y) * Q
        sB = my_y * Q

        a0f = xfer(0, x_ref.at[pl.ds(aH_peer + fA, Q), :],
                   rA0.at[pl.ds(fA, Q), :], y_nbr)
        a0s = xfer(1, x_ref.at[pl.ds(aH_peer + sA, Q), :],
                   rA0.at[pl.ds(sA, Q), :], y_nbr)
        b0f = xfer(2, x_ref.at[pl.ds(bH_peer + fB, Q), :],
                   rB0.at[pl.ds(fB, Q), :], x_nbr)
        b0s = xfer(3, x_ref.at[pl.ds(bH_peer + sB, Q), :],
                   rB0.at[pl.ds(sB, Q), :], x_nbr)
        a0f.start()
        a0s.start()
        b0f.start()
        b0s.start()
        inA = pltpu.make_async_copy(
            x_ref.at[pl.ds(aH_mine, H), :], xAv, local_sems.at[0])
        inB = pltpu.make_async_copy(
            x_ref.at[pl.ds(bH_mine, H), :], xBv, local_sems.at[1])
        inA.start()
        inB.start()

        a1c = [xfer(4 + c, accA.at[pl.ds(fA + c * E, E), :],
                    rA1.at[pl.ds(c * E, E), :], x_nbr) for c in (0, 1)]
        b1c = [xfer(6 + c, accB.at[pl.ds(fB + c * E, E), :],
                    rB1.at[pl.ds(c * E, E), :], y_nbr) for c in (0, 1)]
        inA.wait()
        a0f.wait_recv()
        accA[pl.ds(fA, Q), :] = xAv[pl.ds(fA, Q), :] + rA0[pl.ds(fA, Q), :]
        a1c[0].start()
        a1c[1].start()
        inB.wait()
        b0f.wait_recv()
        accB[pl.ds(fB, Q), :] = xBv[pl.ds(fB, Q), :] + rB0[pl.ds(fB, Q), :]
        b1c[0].start()
        b1c[1].start()

        a2c = [xfer(8 + c, accA.at[pl.ds(sA + c * E, E), :],
                    accA.at[pl.ds(sA + c * E, E), :], x_nbr) for c in (0, 1)]
        b2c = [xfer(10 + c, accB.at[pl.ds(sB + c * E, E), :],
                    accB.at[pl.ds(sB + c * E, E), :], y_nbr) for c in (0, 1)]
        a3s = xfer(12, accA.at[pl.ds(sA, Q), :],
                   out_ref.at[pl.ds(aH_mine + sA, Q), :], y_nbr)
        a3f = xfer(13, accA.at[pl.ds(fA, Q), :],
                   out_ref.at[pl.ds(aH_mine + fA, Q), :], y_nbr)
        b3s = xfer(14, accB.at[pl.ds(sB, Q), :],
                   out_ref.at[pl.ds(bH_mine + sB, Q), :], x_nbr)
        b3f = xfer(15, accB.at[pl.ds(fB, Q), :],
                   out_ref.at[pl.ds(bH_mine + fB, Q), :], x_nbr)

        a0s.wait_recv()
        b0s.wait_recv()
        a1c[0].wait_recv()
        accA[pl.ds(sA, E), :] = (
            xAv[pl.ds(sA, E), :] + rA0[pl.ds(sA, E), :] + rA1[pl.ds(0, E), :]
        )
        a2c[0].start()
        b1c[0].wait_recv()
        accB[pl.ds(sB, E), :] = (
            xBv[pl.ds(sB, E), :] + rB0[pl.ds(sB, E), :] + rB1[pl.ds(0, E), :]
        )
        b2c[0].start()
        a1c[1].wait_recv()
        accA[pl.ds(sA + E, E), :] = (
            xAv[pl.ds(sA + E, E), :] + rA0[pl.ds(sA + E, E), :]
            + rA1[pl.ds(E, E), :]
        )
        a2c[1].start()
        a3s.start()
        b1c[1].wait_recv()
        accB[pl.ds(sB + E, E), :] = (
            xBv[pl.ds(sB + E, E), :] + rB0[pl.ds(sB + E, E), :]
            + rB1[pl.ds(E, E), :]
        )
        b2c[1].start()
        b3s.start()

        outAs = pltpu.make_async_copy(
            accA.at[pl.ds(sA, Q), :],
            out_ref.at[pl.ds(aH_mine + sA, Q), :], local_sems.at[2])
        outBs = pltpu.make_async_copy(
            accB.at[pl.ds(sB, Q), :],
            out_ref.at[pl.ds(bH_mine + sB, Q), :], local_sems.at[3])
        outAs.start()
        outBs.start()

        a2c[0].wait_recv()
        a2c[1].wait_recv()
        a3f.start()
        outAf = pltpu.make_async_copy(
            accA.at[pl.ds(fA, Q), :],
            out_ref.at[pl.ds(aH_mine + fA, Q), :], local_sems.at[4])
        outAf.start()
        b2c[0].wait_recv()
        b2c[1].wait_recv()
        b3f.start()
        outBf = pltpu.make_async_copy(
            accB.at[pl.ds(fB, Q), :],
            out_ref.at[pl.ds(bH_mine + fB, Q), :], local_sems.at[5])
        outBf.start()

        a3s.wait_recv()
        a3f.wait_recv()
        b3s.wait_recv()
        b3f.wait_recv()
        outAs.wait()
        outBs.wait()
        outAf.wait()
        outBf.wait()
        for r in (a0f, a0s, b0f, b0s, *a1c, *b1c, *a2c, *b2c,
                  a3s, a3f, b3s, b3f):
            r.wait_send()

    return pl.pallas_call(
        body,
        out_shape=jax.ShapeDtypeStruct((M, N), jnp.float32),
        in_specs=[pl.BlockSpec(memory_space=pl.ANY)],
        out_specs=pl.BlockSpec(memory_space=pl.ANY),
        scratch_shapes=[
            pltpu.VMEM((H, N), jnp.float32),
            pltpu.VMEM((H, N), jnp.float32),
            pltpu.VMEM((H, N), jnp.float32),
            pltpu.VMEM((H, N), jnp.float32),
            pltpu.VMEM((H, N), jnp.float32),
            pltpu.VMEM((Q, N), jnp.float32),
            pltpu.VMEM((H, N), jnp.float32),
            pltpu.VMEM((Q, N), jnp.float32),
            pltpu.SemaphoreType.DMA((16,)),
            pltpu.SemaphoreType.DMA((16,)),
            pltpu.SemaphoreType.DMA((6,)),
        ],
        compiler_params=pltpu.CompilerParams(collective_id=0),
    )(x)
